# baseline (speedup 1.0000x reference)
import functools

import jax
import jax.numpy as jnp
from jax import lax
from jax.experimental import pallas as pl
from jax.experimental.pallas import tpu as pltpu
from jax.experimental.pallas import tpu_sc as plsc

_HI = lax.Precision.HIGHEST


def _sc_counts(ei_flat, zrow, n, e):
    info = plsc.get_sparse_core_info()
    nc, ns, L = info.num_cores, info.num_subcores, info.num_lanes
    nw = nc * ns
    rows_w = n // nw
    words_w = rows_w * n
    mesh = plsc.VectorSubcoreMesh(core_axis_name="c", subcore_axis_name="s")

    half = e // 2
    trash = words_w

    @functools.partial(
        pl.kernel,
        out_type=jax.ShapeDtypeStruct((n * n,), jnp.float32),
        mesh=mesh,
        compiler_params=pltpu.CompilerParams(needs_layout_passes=False),
        scratch_types=[
            pltpu.VMEM((half,), jnp.int32),
            pltpu.VMEM((half,), jnp.int32),
            pltpu.VMEM((half + L,), jnp.int32),
            pltpu.VMEM((words_w + L,), jnp.float32),
        ],
    )
    def k(ei_hbm, z_hbm, out_hbm, src_v, dst_v, buf_v, c_v):
        wid = lax.axis_index("s") * nc + lax.axis_index("c")
        row0 = wid * rows_w
        pltpu.sync_copy(z_hbm, c_v.at[pl.ds(0, words_w)])
        ones = jnp.ones((L,), jnp.float32)
        base = row0 * n
        bound = jnp.uint32(words_w)
        trash_vec = jnp.full((L,), trash, jnp.int32)
        true_m = jnp.ones((L,), jnp.bool_)

        for p in range(2):
            pltpu.sync_copy(ei_hbm.at[pl.ds(p * half, half)], src_v)
            pltpu.sync_copy(ei_hbm.at[pl.ds(e + p * half, half)], dst_v)

            def cbody(i, ofs):
                s = src_v[pl.ds(i, L)]
                d = dst_v[pl.ds(i, L)]
                lcl = d * n + s - base
                m = lcl.astype(jnp.uint32) < bound
                plsc.store_compressed(buf_v.at[pl.ds(ofs, L)], lcl, mask=m)
                return ofs + jnp.sum(m.astype(jnp.int32))

            ofs = plsc.parallel_loop(0, half, L, unroll=8,
                                     carry=jnp.int32(0))(cbody)
            plsc.store_compressed(buf_v.at[pl.ds(ofs, L)], trash_vec, mask=true_m)

            ng = (ofs + L - 1) // L

            def sbody(g, carry):
                v = buf_v[pl.ds(g * L, L)]
                plsc.addupdate_scatter(c_v, [v], ones)
                return carry

            lax.fori_loop(0, ng, sbody, 0)

        iota = lax.iota(jnp.int32, L)
        for r0 in range(0, rows_w, L):
            rr = iota + r0
            plsc.addupdate_scatter(c_v, [rr * (n + 1) + row0], ones)

        pltpu.sync_copy(c_v.at[pl.ds(0, words_w)],
                        out_hbm.at[pl.ds(row0 * n, words_w)])

    return k(ei_flat, zrow)


def _cdot(c3_ref, t):
    t_hi = t.astype(jnp.bfloat16)
    t_lo = (t - t_hi.astype(jnp.float32)).astype(jnp.bfloat16)
    acc = None
    for s in range(8):
        cs = c3_ref[:, s, :].astype(jnp.bfloat16)
        th = t_hi[s * 128:(s + 1) * 128]
        tl = t_lo[s * 128:(s + 1) * 128]
        p = (jnp.dot(cs, th, preferred_element_type=jnp.float32)
             + jnp.dot(cs, tl, preferred_element_type=jnp.float32))
        acc = p if acc is None else acc + p
    return acc


def _tc_dense_body(x_ref, c3_ref, w1_ref, b1_ref, w2_ref, b2_ref,
                   we1lo_ref, we1hi_ref, be1_ref,
                   node_ref, ap_ref, bpt_ref):
    deg = None
    for s in range(8):
        ds_ = jnp.sum(c3_ref[:, s, :], axis=1, keepdims=True)
        deg = ds_ if deg is None else deg + ds_
    dinv = jnp.where(deg > 0, lax.rsqrt(deg), 0.0)

    xw1 = jnp.dot(x_ref[...], w1_ref[...],
                  preferred_element_type=jnp.float32, precision=_HI)
    h = jnp.maximum(dinv * _cdot(c3_ref, dinv * xw1) + b1_ref[...], 0.0)

    xw2 = jnp.dot(h, w2_ref[...],
                  preferred_element_type=jnp.float32, precision=_HI)
    node_ref[...] = dinv * _cdot(c3_ref, dinv * xw2) + b2_ref[...]

    ap_ref[...] = jnp.dot(h, we1lo_ref[...],
                          preferred_element_type=jnp.float32,
                          precision=_HI) + be1_ref[...]
    bpt_ref[...] = lax.dot_general(
        we1hi_ref[...], h, (((0,), (1,)), ((), ())),
        preferred_element_type=jnp.float32, precision=_HI)


def _tc_dense(x, c3, w1, b1r, w2, b2r, we1lo, we1hi, be1r, n, interpret=False):
    return pl.pallas_call(
        _tc_dense_body,
        out_shape=(
            jax.ShapeDtypeStruct((n, 2), jnp.float32),
            jax.ShapeDtypeStruct((n, 16), jnp.float32),
            jax.ShapeDtypeStruct((16, n), jnp.float32),
        ),
        interpret=interpret,
    )(x, c3, w1, b1r, w2, b2r, we1lo, we1hi, be1r)


def _tc_pairs_body(ap_ref, bpt_ref, we2_ref, be2_ref, eo_ref, *, br, n):
    a = ap_ref[...]
    bt = bpt_ref[...]
    acc = jnp.zeros((br, n), jnp.float32)
    for k in range(16):
        zk = jnp.maximum(a[:, k:k + 1] + bt[k:k + 1, :], 0.0)
        acc = acc + zk * we2_ref[0, k]
    val = 1.0 / (1.0 + jnp.exp(-(acc + be2_ref[0, 0])))
    eo_ref[...] = val.reshape(br, 8, 128)


def _tc_pairs(ap, bpt, we2r, be2r, n, br, interpret=False):
    grid = n // br
    return pl.pallas_call(
        functools.partial(_tc_pairs_body, br=br, n=n),
        grid=(grid,),
        in_specs=[
            pl.BlockSpec((br, 16), lambda i: (i, 0)),
            pl.BlockSpec((16, n), lambda i: (0, 0)),
            pl.BlockSpec(memory_space=pltpu.SMEM),
            pl.BlockSpec(memory_space=pltpu.SMEM),
        ],
        out_specs=pl.BlockSpec((br, 8, 128), lambda i: (i, 0, 0)),
        out_shape=jax.ShapeDtypeStruct((n, 8, 128), jnp.float32),
        interpret=interpret,
    )(ap, bpt, we2r, be2r)


def _tc_fei_body(fei_ref, *, br, n):
    blk = pl.program_id(0)
    jidx = (lax.broadcasted_iota(jnp.int32, (br, 8, 128), 1) * 128
            + lax.broadcasted_iota(jnp.int32, (br, 8, 128), 2))
    fei_ref[0] = (lax.broadcasted_iota(jnp.int32, (br, 8, 128), 0)
                  + blk * br)
    fei_ref[1] = jidx


def _tc_fei(n, br, interpret=False):
    return pl.pallas_call(
        functools.partial(_tc_fei_body, br=br, n=n),
        grid=(n // br,),
        out_specs=pl.BlockSpec((2, br, 8, 128), lambda i: (0, i, 0, 0)),
        out_shape=jax.ShapeDtypeStruct((2, n, 8, 128), jnp.int32),
        interpret=interpret,
    )()


def kernel(x, edge_index, W1, b1, W2, b2, We1, be1, We2, be2):
    n = x.shape[0]
    e = edge_index.shape[1]

    c3 = _sc_counts(edge_index.reshape(-1),
                    jnp.zeros((n * n // 32,), jnp.float32), n, e
                    ).reshape(n, 8, 128)

    node_out, ap, bpt = _tc_dense(
        x, c3, W1, b1.reshape(1, 16), W2, b2.reshape(1, 2),
        We1[:16], We1[16:], be1.reshape(1, 16), n)

    eo3 = _tc_pairs(ap, bpt, We2.reshape(1, 16),
                    be2.reshape(1, 1), n, br=128)
    fei = _tc_fei(n, br=256)

    return node_out, eo3.reshape(n * n), fei.reshape(2, n * n)


# speedup vs baseline: 1.0397x; 1.0397x over previous
import functools

import jax
import jax.numpy as jnp
from jax import lax
from jax.experimental import pallas as pl
from jax.experimental.pallas import tpu as pltpu
from jax.experimental.pallas import tpu_sc as plsc

_HI = lax.Precision.HIGHEST


def _sc_counts(ei_flat, zrow, n, e):
    info = plsc.get_sparse_core_info()
    nc, ns, L = info.num_cores, info.num_subcores, info.num_lanes
    nw = nc * ns
    rows_w = n // nw
    words_w = rows_w * n
    mesh = plsc.VectorSubcoreMesh(core_axis_name="c", subcore_axis_name="s")

    @functools.partial(
        pl.kernel,
        out_type=jax.ShapeDtypeStruct((n * n,), jnp.float32),
        mesh=mesh,
        compiler_params=pltpu.CompilerParams(needs_layout_passes=False),
        scratch_types=[
            pltpu.VMEM((e,), jnp.int32),
            pltpu.VMEM((e,), jnp.int32),
            pltpu.VMEM((words_w,), jnp.float32),
        ],
    )
    def k(ei_hbm, z_hbm, out_hbm, src_v, dst_v, c_v):
        wid = lax.axis_index("s") * nc + lax.axis_index("c")
        row0 = wid * rows_w
        pltpu.sync_copy(ei_hbm.at[pl.ds(0, e)], src_v)
        pltpu.sync_copy(ei_hbm.at[pl.ds(e, e)], dst_v)
        pltpu.sync_copy(z_hbm, c_v)
        ones = jnp.ones((L,), jnp.float32)
        base = row0 * n
        bound = jnp.uint32(words_w)

        @plsc.parallel_loop(0, e, L, unroll=8)
        def ebody(i):
            s = src_v[pl.ds(i, L)]
            d = dst_v[pl.ds(i, L)]
            lcl = d * n + s - base
            m = lcl.astype(jnp.uint32) < bound
            plsc.addupdate_scatter(c_v, [lcl], ones, mask=m)

        iota = lax.iota(jnp.int32, L)
        for r0 in range(0, rows_w, L):
            rr = iota + r0
            plsc.addupdate_scatter(c_v, [rr * (n + 1) + row0], ones)

        pltpu.sync_copy(c_v, out_hbm.at[pl.ds(row0 * n, words_w)])

    return k(ei_flat, zrow)


def _cdot(c3_ref, t):
    t_hi = t.astype(jnp.bfloat16)
    t_lo = (t - t_hi.astype(jnp.float32)).astype(jnp.bfloat16)
    acc = None
    for s in range(8):
        cs = c3_ref[:, s, :].astype(jnp.bfloat16)
        th = t_hi[s * 128:(s + 1) * 128]
        tl = t_lo[s * 128:(s + 1) * 128]
        p = (jnp.dot(cs, th, preferred_element_type=jnp.float32)
             + jnp.dot(cs, tl, preferred_element_type=jnp.float32))
        acc = p if acc is None else acc + p
    return acc


def _tc_dense_body(x_ref, c3_ref, w1_ref, b1_ref, w2_ref, b2_ref,
                   we1lo_ref, we1hi_ref, be1_ref,
                   node_ref, ap_ref, bpt_ref):
    deg = None
    for s in range(8):
        ds_ = jnp.sum(c3_ref[:, s, :], axis=1, keepdims=True)
        deg = ds_ if deg is None else deg + ds_
    dinv = jnp.where(deg > 0, lax.rsqrt(deg), 0.0)

    xw1 = jnp.dot(x_ref[...], w1_ref[...],
                  preferred_element_type=jnp.float32, precision=_HI)
    h = jnp.maximum(dinv * _cdot(c3_ref, dinv * xw1) + b1_ref[...], 0.0)

    xw2 = jnp.dot(h, w2_ref[...],
                  preferred_element_type=jnp.float32, precision=_HI)
    node_ref[...] = dinv * _cdot(c3_ref, dinv * xw2) + b2_ref[...]

    ap_ref[...] = jnp.dot(h, we1lo_ref[...],
                          preferred_element_type=jnp.float32,
                          precision=_HI) + be1_ref[...]
    bpt_ref[...] = lax.dot_general(
        we1hi_ref[...], h, (((0,), (1,)), ((), ())),
        preferred_element_type=jnp.float32, precision=_HI)


def _tc_dense(x, c3, w1, b1r, w2, b2r, we1lo, we1hi, be1r, n, interpret=False):
    return pl.pallas_call(
        _tc_dense_body,
        out_shape=(
            jax.ShapeDtypeStruct((n, 2), jnp.float32),
            jax.ShapeDtypeStruct((n, 16), jnp.float32),
            jax.ShapeDtypeStruct((16, n), jnp.float32),
        ),
        interpret=interpret,
    )(x, c3, w1, b1r, w2, b2r, we1lo, we1hi, be1r)


def _tc_pairs_body(ap_ref, bpt_ref, we2_ref, be2_ref, eo_ref, *, br, n):
    a = ap_ref[...]
    bt = bpt_ref[...]
    acc = jnp.zeros((br, n), jnp.float32)
    for k in range(16):
        zk = jnp.maximum(a[:, k:k + 1] + bt[k:k + 1, :], 0.0)
        acc = acc + zk * we2_ref[0, k]
    val = 1.0 / (1.0 + jnp.exp(-(acc + be2_ref[0, 0])))
    eo_ref[...] = val.reshape(br, 8, 128)


def _tc_pairs(ap, bpt, we2r, be2r, n, br, interpret=False):
    grid = n // br
    return pl.pallas_call(
        functools.partial(_tc_pairs_body, br=br, n=n),
        grid=(grid,),
        in_specs=[
            pl.BlockSpec((br, 16), lambda i: (i, 0)),
            pl.BlockSpec((16, n), lambda i: (0, 0)),
            pl.BlockSpec(memory_space=pltpu.SMEM),
            pl.BlockSpec(memory_space=pltpu.SMEM),
        ],
        out_specs=pl.BlockSpec((br, 8, 128), lambda i: (i, 0, 0)),
        out_shape=jax.ShapeDtypeStruct((n, 8, 128), jnp.float32),
        interpret=interpret,
    )(ap, bpt, we2r, be2r)


def _tc_fei_body(fei_ref, *, br, n):
    blk = pl.program_id(0)
    jidx = (lax.broadcasted_iota(jnp.int32, (br, 8, 128), 1) * 128
            + lax.broadcasted_iota(jnp.int32, (br, 8, 128), 2))
    fei_ref[0] = (lax.broadcasted_iota(jnp.int32, (br, 8, 128), 0)
                  + blk * br)
    fei_ref[1] = jidx


def _tc_fei(n, br, interpret=False):
    return pl.pallas_call(
        functools.partial(_tc_fei_body, br=br, n=n),
        grid=(n // br,),
        out_specs=pl.BlockSpec((2, br, 8, 128), lambda i: (0, i, 0, 0)),
        out_shape=jax.ShapeDtypeStruct((2, n, 8, 128), jnp.int32),
        interpret=interpret,
    )()


def kernel(x, edge_index, W1, b1, W2, b2, We1, be1, We2, be2):
    n = x.shape[0]
    e = edge_index.shape[1]

    c3 = _sc_counts(edge_index.reshape(-1),
                    jnp.zeros((n * n // 32,), jnp.float32), n, e
                    ).reshape(n, 8, 128)

    node_out, ap, bpt = _tc_dense(
        x, c3, W1, b1.reshape(1, 16), W2, b2.reshape(1, 2),
        We1[:16], We1[16:], be1.reshape(1, 16), n)

    eo3 = _tc_pairs(ap, bpt, We2.reshape(1, 16),
                    be2.reshape(1, 1), n, br=128)
    fei = _tc_fei(n, br=256)

    return node_out, eo3.reshape(n * n), fei.reshape(2, n * n)
